# baseline (speedup 1.0000x reference)
import functools

import jax
import jax.numpy as jnp
from jax import lax
from jax.experimental import pallas as pl
from jax.experimental.pallas import tpu as pltpu
from jax.experimental.pallas import tpu_sc as plsc

EMB = 128
NSEG = 10000
S = 12.0
NW = 32
SEG_PER_W = 320
NSEG_PAD = NW * SEG_PER_W
GB = 128
SCAN_CHUNK = 2000

_PAD_B, _PAD_U, _PAD_T = 323584, 81920, 122880
_LEN_B, _LEN_U, _LEN_T = 320000, 80000, 120000


def _gather_body(table, relb, relu, relt, outb, outu, outt, idx_v, rows_v, sem):
    wid = lax.axis_index("s") * 2 + lax.axis_index("c")

    def do_rel(rel, out, padlen):
        rows_w = padlen // NW

        def body(c, carry):
            base = wid * rows_w + c * GB
            pltpu.sync_copy(rel.at[pl.ds(base, GB)], idx_v)
            pltpu.async_copy(table.at[idx_v], rows_v, sem).wait()
            pltpu.sync_copy(rows_v, out.at[pl.ds(base, GB)])
            return carry

        lax.fori_loop(0, rows_w // GB, body, 0)

    do_rel(relb, outb, _PAD_B)
    do_rel(relu, outu, _PAD_U)
    do_rel(relt, outt, _PAD_T)


_gather_call = pl.kernel(
    _gather_body,
    out_type=(
        jax.ShapeDtypeStruct((_PAD_B, EMB), jnp.float32),
        jax.ShapeDtypeStruct((_PAD_U, EMB), jnp.float32),
        jax.ShapeDtypeStruct((_PAD_T, EMB), jnp.float32),
    ),
    mesh=plsc.VectorSubcoreMesh(core_axis_name="c", subcore_axis_name="s"),
    compiler_params=pltpu.CompilerParams(needs_layout_passes=False),
    scratch_types=[
        pltpu.VMEM((GB,), jnp.int32),
        pltpu.VMEM((GB, EMB), jnp.float32),
        pltpu.SemaphoreType.DMA,
    ],
)


def _reduce_body(msgb, msgu, msgt, relb, relu, relt, m_out, s_out,
                 macc, sacc, idxbuf, rowbuf, destbuf, rows_v, sem):
    wid = lax.axis_index("s") * 2 + lax.axis_index("c")
    lo = wid * SEG_PER_W
    iota16 = lax.iota(jnp.int32, 16)

    def initrow(q, carry):
        for j in range(8):
            sl = pl.ds(j * 16, 16)
            macc[q, sl] = jnp.full((16,), -3.0e38, jnp.float32)
            sacc[q, sl] = jnp.full((16,), 1e-16, jnp.float32)
        return carry

    lax.fori_loop(0, SEG_PER_W, initrow, 0)

    def scan_rel(msgs, rel, flat_len, phase):
        def zb(i, carry):
            rowbuf[pl.ds(i * 16, 16)] = jnp.zeros((16,), jnp.int32)
            return carry

        lax.fori_loop(0, 2048 // 16, zb, 0)

        def chunk_body(c, carry):
            pltpu.sync_copy(rel.at[pl.ds(c * SCAN_CHUNK, SCAN_CHUNK)], idxbuf)
            cbase = c * SCAN_CHUNK

            def group_body(g, ptr):
                iv = idxbuf[pl.ds(g * 16, 16)]
                rid = (cbase + g * 16) + iota16
                mask = (iv >= lo) & (iv < lo + SEG_PER_W)
                cs = plsc.cumsum(jnp.where(mask, 1, 0))
                ofs = ptr + cs - 1
                plsc.store_scatter(rowbuf, [ofs], rid, mask=mask)
                plsc.store_scatter(destbuf, [ofs], iv - lo, mask=mask)
                return ptr + cs[15]

            ptr = lax.fori_loop(0, SCAN_CHUNK // 16, group_body, 0)

            def batch_body(k, carry):
                pltpu.async_copy(msgs.at[rowbuf.at[pl.ds(k * GB, GB)]],
                                 rows_v, sem).wait()
                nrows = jnp.minimum(GB, ptr - k * GB)

                def acc_body(r, c2):
                    q = destbuf[pl.ds(k * GB + r, 16)][0]
                    for j in range(8):
                        sl = pl.ds(j * 16, 16)
                        xv = rows_v[r, sl]
                        if phase == 0:
                            macc[q, sl] = jnp.maximum(macc[q, sl], xv)
                        else:
                            sacc[q, sl] = sacc[q, sl] + jnp.exp(
                                S * (xv - macc[q, sl]))
                    return c2

                lax.fori_loop(0, nrows, acc_body, 0)
                return carry

            lax.fori_loop(0, (ptr + GB - 1) // GB, batch_body, 0)
            return carry

        lax.fori_loop(0, flat_len // SCAN_CHUNK, chunk_body, 0)

    scan_rel(msgb, relb, _LEN_B, 0)
    scan_rel(msgu, relu, _LEN_U, 0)
    scan_rel(msgt, relt, _LEN_T, 0)

    def clean(q, carry):
        for j in range(8):
            sl = pl.ds(j * 16, 16)
            v = macc[q, sl]
            macc[q, sl] = jnp.where(v < -1.0e38,
                                    jnp.zeros((16,), jnp.float32), v)
        return carry

    lax.fori_loop(0, SEG_PER_W, clean, 0)

    scan_rel(msgb, relb, _LEN_B, 1)
    scan_rel(msgu, relu, _LEN_U, 1)
    scan_rel(msgt, relt, _LEN_T, 1)

    pltpu.sync_copy(macc, m_out.at[pl.ds(lo, SEG_PER_W)])
    pltpu.sync_copy(sacc, s_out.at[pl.ds(lo, SEG_PER_W)])


_reduce_call = pl.kernel(
    _reduce_body,
    out_type=(
        jax.ShapeDtypeStruct((NSEG_PAD, EMB), jnp.float32),
        jax.ShapeDtypeStruct((NSEG_PAD, EMB), jnp.float32),
    ),
    mesh=plsc.VectorSubcoreMesh(core_axis_name="c", subcore_axis_name="s"),
    compiler_params=pltpu.CompilerParams(needs_layout_passes=False),
    scratch_types=[
        pltpu.VMEM((SEG_PER_W, EMB), jnp.float32),
        pltpu.VMEM((SEG_PER_W, EMB), jnp.float32),
        pltpu.VMEM((SCAN_CHUNK,), jnp.int32),
        pltpu.VMEM((2048,), jnp.int32),
        pltpu.VMEM((2048,), jnp.int32),
        pltpu.VMEM((GB, EMB), jnp.float32),
        pltpu.SemaphoreType.DMA,
    ],
)


def _mish(x):
    sp = jnp.maximum(x, 0.0) + jnp.log1p(jnp.exp(-jnp.abs(x)))
    return x * jnp.tanh(sp)


def _mlp_block_kernel(x_ref, wi_ref, bi_ref, wo_ref, bo_ref, o_ref):
    x = x_ref[...]
    pre = jnp.dot(x, wi_ref[...], preferred_element_type=jnp.float32) + bi_ref[...]
    act = _mish(pre)
    o_ref[...] = x + jnp.dot(act, wo_ref[...],
                             preferred_element_type=jnp.float32) + bo_ref[...]


def _rel_mlp(x, p, br=512):
    e, d = x.shape
    return pl.pallas_call(
        _mlp_block_kernel,
        grid=(e // br,),
        in_specs=[
            pl.BlockSpec((br, d), lambda i: (i, 0)),
            pl.BlockSpec((d, d), lambda i: (0, 0)),
            pl.BlockSpec((1, d), lambda i: (0, 0)),
            pl.BlockSpec((d, d), lambda i: (0, 0)),
            pl.BlockSpec((1, d), lambda i: (0, 0)),
        ],
        out_specs=pl.BlockSpec((br, d), lambda i: (i, 0)),
        out_shape=jax.ShapeDtypeStruct((e, d), jnp.float32),
    )(x, p["Wi"].T, p["bi"][None, :], p["Wo"].T, p["bo"][None, :])


def _update_kernel(m_ref, s_ref, obj_ref, wi_ref, bi_ref, wo_ref, bo_ref, o_ref):
    mm = jnp.log(s_ref[...]) * (1.0 / S) + m_ref[...]
    x = jnp.concatenate([mm, obj_ref[...]], axis=1)
    pre = jnp.dot(x, wi_ref[...], preferred_element_type=jnp.float32) + bi_ref[...]
    act = _mish(pre)
    o_ref[...] = obj_ref[...] + jnp.dot(act, wo_ref[...],
                                        preferred_element_type=jnp.float32) + bo_ref[...]


def _update(m, s, obj, p, br=400):
    return pl.pallas_call(
        _update_kernel,
        grid=(NSEG // br,),
        in_specs=[
            pl.BlockSpec((br, EMB), lambda i: (i, 0)),
            pl.BlockSpec((br, EMB), lambda i: (i, 0)),
            pl.BlockSpec((br, EMB), lambda i: (i, 0)),
            pl.BlockSpec((2 * EMB, 2 * EMB), lambda i: (0, 0)),
            pl.BlockSpec((1, 2 * EMB), lambda i: (0, 0)),
            pl.BlockSpec((2 * EMB, EMB), lambda i: (0, 0)),
            pl.BlockSpec((1, EMB), lambda i: (0, 0)),
        ],
        out_specs=pl.BlockSpec((br, EMB), lambda i: (i, 0)),
        out_shape=jax.ShapeDtypeStruct((NSEG, EMB), jnp.float32),
    )(m, s, obj, p["Wi"].T, p["bi"][None, :], p["Wo"].T, p["bo"][None, :])


def kernel(object_embeddings, rel_binary, rel_unary, rel_ternary, params):
    relb = jnp.concatenate([rel_binary, jnp.zeros((_PAD_B - _LEN_B,), jnp.int32)])
    relu = jnp.concatenate([rel_unary, jnp.zeros((_PAD_U - _LEN_U,), jnp.int32)])
    relt = jnp.concatenate([rel_ternary, jnp.zeros((_PAD_T - _LEN_T,), jnp.int32)])

    gb, gu, gt = _gather_call(object_embeddings, relb, relu, relt)

    mb = _rel_mlp(gb.reshape(_PAD_B // 2, 2 * EMB), params["binary"])
    mu = _rel_mlp(gu, params["unary"])
    mt = _rel_mlp(gt.reshape(_PAD_T // 3, 3 * EMB), params["ternary"])

    m, s = _reduce_call(
        mb.reshape(_PAD_B, EMB), mu, mt.reshape(_PAD_T, EMB),
        rel_binary, rel_unary, rel_ternary)

    return _update(m, s, object_embeddings, params["update"])
